# Initial kernel scaffold; baseline (speedup 1.0000x reference)
import jax
import jax.numpy as jnp
from jax.experimental import pallas as pl


def kernel(pos, z, batch, edge_index, emb, W1, b1, W2, b2, Wr1, br1, Wr2, br2):
    raise NotImplementedError("write your pallas kernel here")


# speedup vs baseline: 17.7924x; 17.7924x over previous
import functools

import jax
import jax.numpy as jnp
from jax import lax
from jax.experimental import pallas as pl
from jax.experimental.pallas import tpu as pltpu
from jax.experimental.pallas import tpu_sc as plsc

N = 10000
E = 320000
H = 128
G = 16

NC = 2
NS = 16
NW = NC * NS
EW = E // NW
CK = 80
NCH = EW // CK
RPS = N // NS

def _sc_degree_body(dst_hbm, out_hbm, dst_v, ones_v, zb_v, hist_sh):
    c = lax.axis_index("c")
    s = lax.axis_index("s")
    w = s * NC + c

    for j in range(CK // 16):
        ones_v[pl.ds(j * 16, 16)] = jnp.ones((16,), jnp.float32)
    for j in range(2000 // 16):
        zb_v[pl.ds(j * 16, 16)] = jnp.zeros((16,), jnp.float32)

    @pl.when(s < 5)
    def _():
        pltpu.sync_copy(zb_v, hist_sh.at[pl.ds(s * 2000, 2000)])

    plsc.subcore_barrier()

    pltpu.sync_copy(dst_hbm.at[w], dst_v)

    def body(j, carry):
        pltpu.sync_copy(ones_v, hist_sh.at[dst_v.at[j]], add=True)
        return carry

    lax.fori_loop(0, NCH, body, 0)
    plsc.subcore_barrier()

    @pl.when(s == 0)
    def _():
        pltpu.sync_copy(hist_sh, out_hbm.at[c, 0, :])


def _sc_edge_agg_body(y_hbm, src_hbm, dst_hbm, out_hbm, src_v, dst_v, rows_v,
                      zb_v, sem, acc_sh):
    c = lax.axis_index("c")
    s = lax.axis_index("s")
    w = s * NC + c

    if True:
        for i in range(16):
            for j in range(H // 16):
                zb_v[i, pl.ds(j * 16, 16)] = jnp.zeros((16,), jnp.float32)

        base = s * 624
        nz = jnp.where(s == NS - 1, 40, 39)

        def zbody(t, carry):
            pltpu.sync_copy(zb_v, acc_sh.at[pl.ds(base + t * 16, 16), :])
            return carry

        lax.fori_loop(0, nz, zbody, 0)

        plsc.subcore_barrier()

        pltpu.sync_copy(src_hbm.at[w], src_v)
        pltpu.sync_copy(dst_hbm.at[w], dst_v)

        def body(j, carry):
            pltpu.async_copy(y_hbm.at[src_v.at[j]], rows_v, sem).wait()
            pltpu.sync_copy(rows_v, acc_sh.at[dst_v.at[j]], add=True)
            return carry

        lax.fori_loop(0, NCH, body, 0)
        plsc.subcore_barrier()

        @pl.when(s < NS - 1)
        def _():
            pltpu.sync_copy(acc_sh.at[pl.ds(base, 624), :],
                            out_hbm.at[c, pl.ds(base, 624), :])

        @pl.when(s == NS - 1)
        def _():
            pltpu.sync_copy(acc_sh.at[pl.ds(15 * 624, 640), :],
                            out_hbm.at[c, pl.ds(15 * 624, 640), :])


@functools.cache
def _sc_kernels():
    mesh = plsc.VectorSubcoreMesh(
        core_axis_name="c", subcore_axis_name="s",
        num_cores=NC, num_subcores=NS)
    sc_degree = pl.kernel(
        _sc_degree_body,
        out_type=jax.ShapeDtypeStruct((NC, 1, N), jnp.float32),
        mesh=mesh,
        scratch_types=[
            pltpu.VMEM((NCH, CK), jnp.int32),
            pltpu.VMEM((CK,), jnp.float32),
            pltpu.VMEM((2000,), jnp.float32),
            pltpu.VMEM_SHARED((N,), jnp.float32),
        ])
    sc_edge_agg = pl.kernel(
        _sc_edge_agg_body,
        out_type=jax.ShapeDtypeStruct((NC, N, H), jnp.float32),
        mesh=mesh,
        scratch_types=[
            pltpu.VMEM((NCH, CK), jnp.int32),
            pltpu.VMEM((NCH, CK), jnp.int32),
            pltpu.VMEM((CK, H), jnp.float32),
            pltpu.VMEM((16, H), jnp.float32),
            pltpu.SemaphoreType.DMA,
            pltpu.VMEM_SHARED((N, H), jnp.float32),
        ])
    return sc_degree, sc_edge_agg


_R = 1000


def _tc1_body(z_ref, d0_ref, d1_ref, emb_ref, w1_ref, y_ref, dinv_ref):
    deg = d0_ref[...] + d1_ref[...] + 1.0
    dinv = lax.rsqrt(deg)
    z = z_ref[...]
    onehot = (z == lax.broadcasted_iota(jnp.int32, (_R, 128), 1)).astype(
        jnp.float32)
    embw = jnp.dot(emb_ref[...], w1_ref[...],
                   preferred_element_type=jnp.float32)
    xw = jnp.dot(onehot, embw, preferred_element_type=jnp.float32)
    y_ref[...] = dinv * xw
    dinv_ref[...] = dinv


def _tc2_body(a0_ref, a1_ref, y_ref, dinv_ref, b_ref, w_ref, out_ref):
    dinv = dinv_ref[...]
    x = dinv * (a0_ref[...] + a1_ref[...] + y_ref[...]) + b_ref[...]
    x = jnp.maximum(x, 0.0)
    out_ref[...] = dinv * jnp.dot(x, w_ref[...],
                                  preferred_element_type=jnp.float32)


def _tc3_body(a0_ref, a1_ref, y_ref, dinv_ref, b_ref, batch_ref, wr1_ref,
              br1_ref, wr2_ref, br2_ref, out_ref, acc_ref, cnt_ref):
    i = pl.program_id(0)
    dinv = dinv_ref[...]
    x = dinv * (a0_ref[...] + a1_ref[...] + y_ref[...]) + b_ref[...]
    x = jnp.maximum(x, 0.0)
    p = (batch_ref[...] == lax.broadcasted_iota(jnp.int32, (_R, G), 1)
         ).astype(jnp.float32)
    dn = (((0,), (0,)), ((), ()))
    psum = lax.dot_general(p, x, dn, preferred_element_type=jnp.float32)
    pcnt = lax.dot_general(p, jnp.ones((_R, 1), jnp.float32), dn,
                           preferred_element_type=jnp.float32)

    @pl.when(i == 0)
    def _():
        acc_ref[...] = jnp.zeros_like(acc_ref)
        cnt_ref[...] = jnp.zeros_like(cnt_ref)

    acc_ref[...] += psum
    cnt_ref[...] += pcnt

    @pl.when(i == pl.num_programs(0) - 1)
    def _():
        pooled = acc_ref[...] / jnp.maximum(cnt_ref[...], 1.0)
        h = jnp.dot(pooled, wr1_ref[...],
                    preferred_element_type=jnp.float32) + br1_ref[...]
        h = h * jax.nn.sigmoid(h)
        out_ref[...] = jnp.dot(h, wr2_ref[...],
                               preferred_element_type=jnp.float32) + br2_ref[...]


def _row_spec(width):
    return pl.BlockSpec((_R, width), lambda i: (i, 0))


def _full_spec(shape):
    return pl.BlockSpec(shape, lambda i: tuple(0 for _ in shape))


def kernel(pos, z, batch, edge_index, emb, W1, b1, W2, b2, Wr1, br1, Wr2, br2):
    del pos
    src3 = edge_index[0].reshape(NW, NCH, CK)
    dst3 = edge_index[1].reshape(NW, NCH, CK)
    z2 = z.reshape(N, 1)
    batch2 = batch.reshape(N, 1)
    embp = jnp.zeros((128, H), jnp.float32).at[:emb.shape[0]].set(emb)
    sc_degree, sc_edge_agg = _sc_kernels()

    deg_part = sc_degree(dst3)
    d0 = deg_part[0, 0].reshape(N, 1)
    d1 = deg_part[1, 0].reshape(N, 1)

    grid = N // _R
    y1, dinv = pl.pallas_call(
        _tc1_body,
        grid=(grid,),
        in_specs=[_row_spec(1), _row_spec(1), _row_spec(1),
                  _full_spec((128, H)), _full_spec((H, H))],
        out_specs=[_row_spec(H), _row_spec(1)],
        out_shape=[jax.ShapeDtypeStruct((N, H), jnp.float32),
                   jax.ShapeDtypeStruct((N, 1), jnp.float32)],
    )(z2, d0, d1, embp, W1)

    agg1 = sc_edge_agg(y1, src3, dst3)

    y2 = pl.pallas_call(
        _tc2_body,
        grid=(grid,),
        in_specs=[_row_spec(H), _row_spec(H), _row_spec(H), _row_spec(1),
                  _full_spec((1, H)), _full_spec((H, H))],
        out_specs=_row_spec(H),
        out_shape=jax.ShapeDtypeStruct((N, H), jnp.float32),
    )(agg1[0], agg1[1], y1, dinv, b1.reshape(1, H), W2)

    agg2 = sc_edge_agg(y2, src3, dst3)

    energy = pl.pallas_call(
        _tc3_body,
        grid=(grid,),
        in_specs=[_row_spec(H), _row_spec(H), _row_spec(H), _row_spec(1),
                  _full_spec((1, H)), _row_spec(1),
                  _full_spec((H, H // 2)), _full_spec((1, H // 2)),
                  _full_spec((H // 2, 1)), _full_spec((1, 1))],
        out_specs=_full_spec((G, 1)),
        out_shape=jax.ShapeDtypeStruct((G, 1), jnp.float32),
        scratch_shapes=[pltpu.VMEM((G, H), jnp.float32),
                        pltpu.VMEM((G, 1), jnp.float32)],
    )(agg2[0], agg2[1], y2, dinv, b2.reshape(1, H), batch2,
      Wr1, br1.reshape(1, H // 2), Wr2, br2.reshape(1, 1))

    return energy.reshape(-1)


# speedup vs baseline: 22.0855x; 1.2413x over previous
import functools

import jax
import jax.numpy as jnp
from jax import lax
from jax.experimental import pallas as pl
from jax.experimental.pallas import tpu as pltpu
from jax.experimental.pallas import tpu_sc as plsc

N = 10000
E = 320000
H = 128
G = 16

NC = 2
NS = 16
NW = NC * NS
EW = E // NW
CK = 80
NCH = EW // CK
DCK = 80
DNCH = EW // DCK
RPS = N // NS

def _sc_degree_body(dst_hbm, out_hbm, dst_v, ones_v, zb_v, hist_sh):
    c = lax.axis_index("c")
    s = lax.axis_index("s")
    w = s * NC + c

    for j in range(DCK // 16):
        ones_v[pl.ds(j * 16, 16)] = jnp.ones((16,), jnp.float32)
    for j in range(2000 // 16):
        zb_v[pl.ds(j * 16, 16)] = jnp.zeros((16,), jnp.float32)

    @pl.when(s < 5)
    def _():
        pltpu.sync_copy(zb_v, hist_sh.at[pl.ds(s * 2000, 2000)])

    plsc.subcore_barrier()

    pltpu.sync_copy(dst_hbm.at[w], dst_v)

    def body(j, carry):
        pltpu.sync_copy(ones_v, hist_sh.at[dst_v.at[j]], add=True)
        return carry

    lax.fori_loop(0, DNCH, body, 0)
    plsc.subcore_barrier()

    @pl.when(s == 0)
    def _():
        pltpu.sync_copy(hist_sh, out_hbm.at[c, 0, :])


def _sc_edge_agg_body(y_hbm, src_hbm, dst_hbm, out_hbm, src_v, dst_v, rows_v,
                      sem0, sem1, acc_sh):
    c = lax.axis_index("c")
    s = lax.axis_index("s")
    w = s * NC + c

    if True:
        for i in range(16):
            for j in range(H // 16):
                rows_v[0, i, pl.ds(j * 16, 16)] = jnp.zeros((16,), jnp.float32)

        base = s * 624
        nz = jnp.where(s == NS - 1, 40, 39)

        def zbody(t, carry):
            pltpu.sync_copy(rows_v.at[0, pl.ds(0, 16), :],
                            acc_sh.at[pl.ds(base + t * 16, 16), :])
            return carry

        lax.fori_loop(0, nz, zbody, 0)

        plsc.subcore_barrier()

        pltpu.sync_copy(src_hbm.at[w], src_v)
        pltpu.sync_copy(dst_hbm.at[w], dst_v)

        pltpu.async_copy(y_hbm.at[src_v.at[pl.ds(0, CK)]], rows_v.at[0], sem0)

        def body(j, carry):
            def step(buf, sem_cur, sem_nxt):
                pltpu.make_async_copy(
                    y_hbm.at[src_v.at[pl.ds(j * CK, CK)]],
                    rows_v.at[buf], sem_cur).wait()

                @pl.when(j < NCH - 1)
                def _():
                    pltpu.async_copy(
                        y_hbm.at[src_v.at[pl.ds((j + 1) * CK, CK)]],
                        rows_v.at[1 - buf], sem_nxt)

                pltpu.sync_copy(rows_v.at[buf], acc_sh.at[dst_v.at[j]],
                                add=True)

            @pl.when(j % 2 == 0)
            def _():
                step(0, sem0, sem1)

            @pl.when(j % 2 == 1)
            def _():
                step(1, sem1, sem0)

            return carry

        lax.fori_loop(0, NCH, body, 0)
        plsc.subcore_barrier()

        @pl.when(s < NS - 1)
        def _():
            pltpu.sync_copy(acc_sh.at[pl.ds(base, 624), :],
                            out_hbm.at[c, pl.ds(base, 624), :])

        @pl.when(s == NS - 1)
        def _():
            pltpu.sync_copy(acc_sh.at[pl.ds(15 * 624, 640), :],
                            out_hbm.at[c, pl.ds(15 * 624, 640), :])


@functools.cache
def _sc_kernels():
    mesh = plsc.VectorSubcoreMesh(
        core_axis_name="c", subcore_axis_name="s",
        num_cores=NC, num_subcores=NS)
    sc_degree = pl.kernel(
        _sc_degree_body,
        out_type=jax.ShapeDtypeStruct((NC, 1, N), jnp.float32),
        mesh=mesh,
        scratch_types=[
            pltpu.VMEM((DNCH, DCK), jnp.int32),
            pltpu.VMEM((DCK,), jnp.float32),
            pltpu.VMEM((2000,), jnp.float32),
            pltpu.VMEM_SHARED((N,), jnp.float32),
        ])
    sc_edge_agg = pl.kernel(
        _sc_edge_agg_body,
        out_type=jax.ShapeDtypeStruct((NC, N, H), jnp.float32),
        mesh=mesh,
        scratch_types=[
            pltpu.VMEM((EW,), jnp.int32),
            pltpu.VMEM((NCH, CK), jnp.int32),
            pltpu.VMEM((2, CK, H), jnp.float32),
            pltpu.SemaphoreType.DMA,
            pltpu.SemaphoreType.DMA,
            pltpu.VMEM_SHARED((N, H), jnp.float32),
        ])
    return sc_degree, sc_edge_agg


_R = 1000


def _tc1_body(z_ref, d0_ref, d1_ref, emb_ref, w1_ref, y_ref, dinv_ref):
    deg = d0_ref[...] + d1_ref[...] + 1.0
    dinv = lax.rsqrt(deg)
    z = z_ref[...]
    onehot = (z == lax.broadcasted_iota(jnp.int32, (_R, 128), 1)).astype(
        jnp.float32)
    embw = jnp.dot(emb_ref[...], w1_ref[...],
                   preferred_element_type=jnp.float32)
    xw = jnp.dot(onehot, embw, preferred_element_type=jnp.float32)
    y_ref[...] = dinv * xw
    dinv_ref[...] = dinv


def _tc2_body(a0_ref, a1_ref, y_ref, dinv_ref, b_ref, w_ref, out_ref):
    dinv = dinv_ref[...]
    x = dinv * (a0_ref[...] + a1_ref[...] + y_ref[...]) + b_ref[...]
    x = jnp.maximum(x, 0.0)
    out_ref[...] = dinv * jnp.dot(x, w_ref[...],
                                  preferred_element_type=jnp.float32)


def _tc3_body(a0_ref, a1_ref, y_ref, dinv_ref, b_ref, batch_ref, wr1_ref,
              br1_ref, wr2_ref, br2_ref, out_ref, acc_ref, cnt_ref):
    i = pl.program_id(0)
    dinv = dinv_ref[...]
    x = dinv * (a0_ref[...] + a1_ref[...] + y_ref[...]) + b_ref[...]
    x = jnp.maximum(x, 0.0)
    p = (batch_ref[...] == lax.broadcasted_iota(jnp.int32, (_R, G), 1)
         ).astype(jnp.float32)
    dn = (((0,), (0,)), ((), ()))
    psum = lax.dot_general(p, x, dn, preferred_element_type=jnp.float32)
    pcnt = lax.dot_general(p, jnp.ones((_R, 1), jnp.float32), dn,
                           preferred_element_type=jnp.float32)

    @pl.when(i == 0)
    def _():
        acc_ref[...] = jnp.zeros_like(acc_ref)
        cnt_ref[...] = jnp.zeros_like(cnt_ref)

    acc_ref[...] += psum
    cnt_ref[...] += pcnt

    @pl.when(i == pl.num_programs(0) - 1)
    def _():
        pooled = acc_ref[...] / jnp.maximum(cnt_ref[...], 1.0)
        h = jnp.dot(pooled, wr1_ref[...],
                    preferred_element_type=jnp.float32) + br1_ref[...]
        h = h * jax.nn.sigmoid(h)
        out_ref[...] = jnp.dot(h, wr2_ref[...],
                               preferred_element_type=jnp.float32) + br2_ref[...]


def _row_spec(width):
    return pl.BlockSpec((_R, width), lambda i: (i, 0))


def _full_spec(shape):
    return pl.BlockSpec(shape, lambda i: tuple(0 for _ in shape))


def kernel(pos, z, batch, edge_index, emb, W1, b1, W2, b2, Wr1, br1, Wr2, br2):
    del pos
    src3 = edge_index[0].reshape(NW, EW)
    dst3 = edge_index[1].reshape(NW, NCH, CK)
    dst3d = edge_index[1].reshape(NW, DNCH, DCK)
    z2 = z.reshape(N, 1)
    batch2 = batch.reshape(N, 1)
    embp = jnp.zeros((128, H), jnp.float32).at[:emb.shape[0]].set(emb)
    sc_degree, sc_edge_agg = _sc_kernels()

    deg_part = sc_degree(dst3d)
    d0 = deg_part[0, 0].reshape(N, 1)
    d1 = deg_part[1, 0].reshape(N, 1)

    grid = N // _R
    y1, dinv = pl.pallas_call(
        _tc1_body,
        grid=(grid,),
        in_specs=[_row_spec(1), _row_spec(1), _row_spec(1),
                  _full_spec((128, H)), _full_spec((H, H))],
        out_specs=[_row_spec(H), _row_spec(1)],
        out_shape=[jax.ShapeDtypeStruct((N, H), jnp.float32),
                   jax.ShapeDtypeStruct((N, 1), jnp.float32)],
    )(z2, d0, d1, embp, W1)

    agg1 = sc_edge_agg(y1, src3, dst3)

    y2 = pl.pallas_call(
        _tc2_body,
        grid=(grid,),
        in_specs=[_row_spec(H), _row_spec(H), _row_spec(H), _row_spec(1),
                  _full_spec((1, H)), _full_spec((H, H))],
        out_specs=_row_spec(H),
        out_shape=jax.ShapeDtypeStruct((N, H), jnp.float32),
    )(agg1[0], agg1[1], y1, dinv, b1.reshape(1, H), W2)

    agg2 = sc_edge_agg(y2, src3, dst3)

    energy = pl.pallas_call(
        _tc3_body,
        grid=(grid,),
        in_specs=[_row_spec(H), _row_spec(H), _row_spec(H), _row_spec(1),
                  _full_spec((1, H)), _row_spec(1),
                  _full_spec((H, H // 2)), _full_spec((1, H // 2)),
                  _full_spec((H // 2, 1)), _full_spec((1, 1))],
        out_specs=_full_spec((G, 1)),
        out_shape=jax.ShapeDtypeStruct((G, 1), jnp.float32),
        scratch_shapes=[pltpu.VMEM((G, H), jnp.float32),
                        pltpu.VMEM((G, 1), jnp.float32)],
    )(agg2[0], agg2[1], y2, dinv, b2.reshape(1, H), batch2,
      Wr1, br1.reshape(1, H // 2), Wr2, br2.reshape(1, 1))

    return energy.reshape(-1)
